# Initial kernel scaffold; baseline (speedup 1.0000x reference)
import jax
import jax.numpy as jnp
from jax.experimental import pallas as pl


def kernel(x, pos, W_word, W_pos):
    raise NotImplementedError("write your pallas kernel here")


# speedup vs baseline: 2.0157x; 2.0157x over previous
import functools

import jax
import jax.numpy as jnp
from jax import lax
from jax.experimental import pallas as pl
from jax.experimental.pallas import tpu as pltpu
from jax.experimental.pallas import tpu_sc as plsc

_B, _L = 4096, 200
_N = _B * _L
_DW, _DP = 128, 16
_D = _DW + _DP
_NC, _NS = 2, 16
_NW = _NC * _NS
_PW = _N // _NW
_G = 128
_NG = _PW // _G


def _build():
  mesh = plsc.VectorSubcoreMesh(core_axis_name="c", subcore_axis_name="s")

  @functools.partial(
      pl.kernel,
      mesh=mesh,
      compiler_params=pltpu.CompilerParams(use_tc_tiling_on_sc=False),
      out_type=jax.ShapeDtypeStruct((_N, _D), jnp.float32),
      scratch_types=[
          pltpu.VMEM((_PW,), jnp.int32),
          pltpu.VMEM((_PW,), jnp.int32),
          pltpu.VMEM((2, _G, _DW), jnp.float32),
          pltpu.VMEM((2, _G, _DP), jnp.float32),
          pltpu.SemaphoreType.DMA,
          pltpu.SemaphoreType.DMA,
          pltpu.SemaphoreType.DMA,
          pltpu.SemaphoreType.DMA,
      ],
  )
  def emb(x_hbm, p_hbm, ww_hbm, wp_hbm, out_hbm,
          xi, pi, wrows, prows, gs0, gs1, ws0, ws1):
    gsem = (gs0, gs1)
    wsem = (ws0, ws1)
    wid = lax.axis_index("s") * _NC + lax.axis_index("c")
    base = wid * _PW
    pltpu.sync_copy(x_hbm.at[pl.ds(base, _PW)], xi)
    pltpu.sync_copy(p_hbm.at[pl.ds(base, _PW)], pi)

    def issue_gather(g, b):
      sl = pl.ds(g * _G, _G)
      pltpu.async_copy(ww_hbm.at[xi.at[sl]], wrows.at[b], gsem[b])
      pltpu.async_copy(wp_hbm.at[pi.at[sl]], prows.at[b], gsem[b])

    def wait_gather(b):
      pltpu.make_async_copy(
          ww_hbm.at[xi.at[pl.ds(0, _G)]], wrows.at[b], gsem[b]).wait()
      pltpu.make_async_copy(
          wp_hbm.at[pi.at[pl.ds(0, _G)]], prows.at[b], gsem[b]).wait()

    def issue_write(g, b):
      row = base + g * _G
      pltpu.async_copy(
          wrows.at[b], out_hbm.at[pl.ds(row, _G), pl.ds(0, _DW)], wsem[b])
      pltpu.async_copy(
          prows.at[b], out_hbm.at[pl.ds(row, _G), pl.ds(_DW, _DP)], wsem[b])

    def wait_write(b):
      pltpu.make_async_copy(
          wrows.at[b], out_hbm.at[pl.ds(0, _G), pl.ds(0, _DW)], wsem[b]).wait()
      pltpu.make_async_copy(
          prows.at[b], out_hbm.at[pl.ds(0, _G), pl.ds(_DW, _DP)], wsem[b]).wait()

    issue_gather(0, 0)
    issue_gather(1, 1)

    @pl.loop(0, _NG, step=2)
    def _groups(g0):
      for b in range(2):
        g = g0 + b
        wait_gather(b)
        issue_write(g, b)
        wait_write(b)

        @pl.when(g + 2 < _NG)
        def _():
          issue_gather(g + 2, b)

  return emb


_emb = _build()


@jax.jit
def kernel(x, pos, W_word, W_pos):
  out = _emb(x.reshape(_N).astype(jnp.int32),
             pos.reshape(_N).astype(jnp.int32),
             W_word, W_pos)
  return out.reshape(_B, _L, _D)
